# Initial kernel scaffold; baseline (speedup 1.0000x reference)
import jax
import jax.numpy as jnp
from jax.experimental import pallas as pl


def kernel(y, table):
    raise NotImplementedError("write your pallas kernel here")


# speedup vs baseline: 4.0912x; 4.0912x over previous
import functools
import jax
import jax.numpy as jnp
from jax import lax
from jax.experimental import pallas as pl
from jax.experimental.pallas import tpu as pltpu
from jax.experimental.pallas import tpu_sc as plsc

K = 100000
M = 64
NC = 2
NS = 16
NW = NC * NS
CH = 128


def _make_lookup(B):
    assert B % (NW * CH) == 0
    n_chunks = B // (NW * CH)
    b_per_w = B // NW
    mesh = plsc.VectorSubcoreMesh(core_axis_name="c", subcore_axis_name="s")

    @functools.partial(
        pl.kernel,
        out_type=jax.ShapeDtypeStruct((B, M), jnp.float32),
        mesh=mesh,
        compiler_params=pltpu.CompilerParams(use_tc_tiling_on_sc=False),
        scratch_types=[
            pltpu.VMEM((n_chunks, CH), jnp.int32),
            pltpu.VMEM((CH, M), jnp.float32),
            pltpu.SemaphoreType.DMA,
        ],
    )
    def lookup(idx_hbm, table_hbm, out_hbm, idx_v, rows_v, sem):
        wid = lax.axis_index("s") * NC + lax.axis_index("c")
        base = wid * b_per_w
        pltpu.sync_copy(idx_hbm.at[wid], idx_v)

        def body(j, carry):
            pltpu.async_copy(table_hbm.at[idx_v.at[j]], rows_v, sem).wait()
            pltpu.sync_copy(rows_v, out_hbm.at[pl.ds(base + j * CH, CH)])
            return carry

        lax.fori_loop(0, n_chunks, body, 0)

    return lookup


def kernel(y, table):
    B = y.shape[0] * y.shape[1]
    idx = y.reshape(NW, B // (NW * CH), CH).astype(jnp.int32)
    out = _make_lookup(B)(idx, table)
    return out.reshape(y.shape[0], y.shape[1], M)


# speedup vs baseline: 4.6357x; 1.1331x over previous
import functools
import jax
import jax.numpy as jnp
from jax import lax
from jax.experimental import pallas as pl
from jax.experimental.pallas import tpu as pltpu
from jax.experimental.pallas import tpu_sc as plsc

K = 100000
M = 64
NC = 2
NS = 16
NW = NC * NS
CH = 128


def _make_lookup(B):
    assert B % (NW * CH) == 0
    n_chunks = B // (NW * CH)
    grp_ch = 5
    n_grp = n_chunks // grp_ch
    assert n_chunks % grp_ch == 0 and n_grp % 2 == 0
    grp_rows = grp_ch * CH
    b_per_w = B // NW
    mesh = plsc.VectorSubcoreMesh(core_axis_name="c", subcore_axis_name="s")

    @functools.partial(
        pl.kernel,
        out_type=jax.ShapeDtypeStruct((B, M), jnp.float32),
        mesh=mesh,
        compiler_params=pltpu.CompilerParams(use_tc_tiling_on_sc=False),
        scratch_types=[
            pltpu.VMEM((n_chunks, CH), jnp.int32),
            pltpu.VMEM((grp_rows, M), jnp.float32),
            pltpu.VMEM((grp_rows, M), jnp.float32),
            pltpu.SemaphoreType.DMA,
            pltpu.SemaphoreType.DMA,
            pltpu.SemaphoreType.DMA,
        ],
    )
    def lookup(idx_hbm, table_hbm, out_hbm, idx_v, rows0, rows1, g0, g1, osem):
        wid = lax.axis_index("s") * NC + lax.axis_index("c")
        base = wid * b_per_w
        pltpu.sync_copy(idx_hbm.at[wid], idx_v)
        bufs = (rows0, rows1)
        gsems = (g0, g1)

        def gathers(cur, b):
            return [
                pltpu.make_async_copy(
                    table_hbm.at[idx_v.at[cur * grp_ch + c]],
                    bufs[b].at[pl.ds(c * CH, CH)],
                    gsems[b],
                )
                for c in range(grp_ch)
            ]

        def stage(g, b):
            cur = g + b
            copies = gathers(cur, b)
            for cp in copies:
                cp.start()

            @pl.when(cur > 0)
            def _():
                out = pltpu.make_async_copy(
                    bufs[1 - b],
                    out_hbm.at[pl.ds(base + (cur - 1) * grp_rows, grp_rows)],
                    osem,
                )
                out.start()
                out.wait()

            for cp in copies:
                cp.wait()

        def body(g):
            stage(g, 0)
            stage(g, 1)

        pl.loop(0, n_grp, step=2)(body)
        pltpu.sync_copy(
            bufs[(n_grp - 1) % 2],
            out_hbm.at[pl.ds(base + (n_grp - 1) * grp_rows, grp_rows)],
        )

    return lookup


def kernel(y, table):
    B = y.shape[0] * y.shape[1]
    idx = y.reshape(NW, B // (NW * CH), CH).astype(jnp.int32)
    out = _make_lookup(B)(idx, table)
    return out.reshape(y.shape[0], y.shape[1], M)
